# baseline (speedup 1.0000x reference)
import functools

import jax
import jax.numpy as jnp
from jax import lax
from jax.experimental import pallas as pl
from jax.experimental.pallas import tpu as pltpu
from jax.experimental.pallas import tpu_sc as plsc

L = 16
NC = 2
NS = 16
NW = NC * NS
SG = 4
QW = 4
NT = 1 << QW


def _make_sc_call(B, CTX, M, LOCAL):
    BW = B // NW
    NG = BW // SG
    NQ = CTX // QW
    MH = M // 2
    MBH = MH // L
    TSTRIDE = NQ * MH

    def body(tq_hbm, ppa_hbm, ppb_hbm, ip_hbm, out_hbm,
             tq_v, pp_v, ip_v, out_v):
        wid = lax.axis_index("c") * NS + lax.axis_index("s")
        base = wid * BW
        pltpu.sync_copy(tq_hbm.at[pl.ds(base, BW)], tq_v)
        pltpu.sync_copy(ip_hbm, ip_v)
        lane_iota = lax.iota(jnp.int32, L)

        for half in range(2):
            pltpu.sync_copy(ppa_hbm if half == 0 else ppb_hbm, pp_v)
            ip_rows = [[ip_v[l, pl.ds(half * MH + j * L, L)]
                        for j in range(MBH)] for l in range(LOCAL)]

            def group_body(g, carry):
                r0 = g * SG

                def chunk_body(qc, acc):
                    q0 = qc * L
                    tqrows = [tq_v[r0 + s, pl.ds(q0, L)] for s in range(SG)]
                    acc = list(acc)
                    for k in range(L):
                        qvec = lane_iota + (q0 + k) * MH
                        for s in range(SG):
                            tb = jnp.broadcast_to(tqrows[s][k], (L,))
                            ib = tb * TSTRIDE + qvec
                            for j in range(MBH):
                                row = plsc.load_gather(pp_v, [ib + j * L])
                                acc[s * MBH + j] = acc[s * MBH + j] * row
                    return tuple(acc)

                init = tuple(jnp.full((L,), 1.0, jnp.float32)
                             for _ in range(SG * MBH))
                acc = lax.fori_loop(0, NQ // L, chunk_body, init)

                carry = list(carry)
                for s in range(SG):
                    lane = (g % (L // SG)) * SG + s
                    for l in range(LOCAL):
                        v = acc[s * MBH] * ip_rows[l][0]
                        for j in range(1, MBH):
                            v = v + acc[s * MBH + j] * ip_rows[l][j]
                        for sh in (8, 4, 2, 1):
                            perm = lane_iota ^ sh
                            v = v + v.at[perm].get(mode="promise_in_bounds")
                        carry[l] = jnp.where(lane_iota == lane, v, carry[l])

                @pl.when(g % (L // SG) == (L // SG) - 1)
                def _():
                    col = (g // (L // SG)) * L
                    for l in range(LOCAL):
                        if half == 0:
                            out_v[l, pl.ds(col, L)] = carry[l]
                        else:
                            out_v[l, pl.ds(col, L)] = (
                                out_v[l, pl.ds(col, L)] + carry[l])

                return tuple(carry)

            zero = jnp.zeros((L,), jnp.float32)
            lax.fori_loop(0, NG, group_body, (zero,) * LOCAL)

        for l in range(LOCAL):
            pltpu.sync_copy(out_v.at[l], out_hbm.at[l, pl.ds(base, BW)])

    mesh = plsc.VectorSubcoreMesh(core_axis_name="c", subcore_axis_name="s")
    return pl.kernel(
        body,
        mesh=mesh,
        compiler_params=pltpu.CompilerParams(needs_layout_passes=False),
        out_type=jax.ShapeDtypeStruct((LOCAL, B), jnp.float32),
        scratch_types=[
            pltpu.VMEM((BW, NQ), jnp.int32),
            pltpu.VMEM((NT * NQ * MH,), jnp.float32),
            pltpu.VMEM((LOCAL, M), jnp.float32),
            pltpu.VMEM((LOCAL, BW), jnp.float32),
        ],
    )


BT = 256
B_SC = 2048


def _make_tc_call(B_tc, CTX, M, LOCAL):
    KC = 128

    def body(ctx_ref, cp0_ref, d_ref, ip_ref, out_ref):
        def chunk(cc, acc):
            c0 = cc * KC
            cols = ctx_ref[:, pl.ds(c0, KC)]
            rows0 = cp0_ref[pl.ds(c0, KC), :]
            rowsd = d_ref[pl.ds(c0, KC), :]
            for k in range(KC):
                colk = lax.slice_in_dim(cols, k, k + 1, axis=1)
                r0k = lax.slice_in_dim(rows0, k, k + 1, axis=0)
                rdk = lax.slice_in_dim(rowsd, k, k + 1, axis=0)
                acc = acc * (r0k + colk * rdk)
            return acc

        acc = lax.fori_loop(0, CTX // KC, chunk,
                            jnp.ones((BT, M), jnp.float32))
        outs = [jnp.sum(acc * ip_ref[pl.ds(l, 1), :], axis=1, keepdims=True)
                for l in range(LOCAL)]
        out_ref[...] = jnp.concatenate(outs, axis=1)

    return pl.pallas_call(
        body,
        grid=(B_tc // BT,),
        in_specs=[
            pl.BlockSpec((BT, CTX), lambda i: (i, 0)),
            pl.BlockSpec((CTX, M), lambda i: (0, 0)),
            pl.BlockSpec((CTX, M), lambda i: (0, 0)),
            pl.BlockSpec((LOCAL, M), lambda i: (0, 0)),
        ],
        out_specs=pl.BlockSpec((BT, LOCAL), lambda i: (i, 0)),
        out_shape=jax.ShapeDtypeStruct((B_tc, LOCAL), jnp.float32),
    )


def kernel(context, context_param, inputs_param):
    LOCAL_N, M, CTX = context_param.shape
    B = context.shape[0]
    NQ = CTX // QW
    b_sc = B_SC if 0 < B_SC < B else B
    ip_f = inputs_param.astype(jnp.float32)

    ctx_i = context.astype(jnp.int32)
    ctx_q = ctx_i[:b_sc].reshape(b_sc, NQ, QW)
    weights = jnp.array([8, 4, 2, 1], jnp.int32)
    tq = jnp.sum(ctx_q * weights, axis=-1).astype(jnp.int32)

    cpT = jnp.transpose(context_param, (0, 2, 1))
    gq = cpT.reshape(LOCAL_N, NQ, QW, M)
    t_idx = jnp.arange(NT)
    pp = (gq[(t_idx >> 3) & 1, :, 0, :]
          * gq[(t_idx >> 2) & 1, :, 1, :]
          * gq[(t_idx >> 1) & 1, :, 2, :]
          * gq[t_idx & 1, :, 3, :])
    ppa = pp[:, :, : M // 2].reshape(-1)
    ppb = pp[:, :, M // 2:].reshape(-1)

    sc_call = _make_sc_call(b_sc, CTX, M, LOCAL_N)
    out_sc = jnp.transpose(sc_call(tq, ppa, ppb, ip_f))
    if b_sc == B:
        return out_sc

    ctx_tc = ctx_i[b_sc:].astype(jnp.float32)
    cp0T = cpT[0]
    dT = cpT[1] - cpT[0]
    tc_call = _make_tc_call(B - b_sc, CTX, M, LOCAL_N)
    out_tc = tc_call(ctx_tc, cp0T, dT, ip_f)

    return jnp.concatenate([out_sc, out_tc], axis=0)


# speedup vs baseline: 1.0821x; 1.0821x over previous
import functools

import jax
import jax.numpy as jnp
from jax import lax
from jax.experimental import pallas as pl
from jax.experimental.pallas import tpu as pltpu
from jax.experimental.pallas import tpu_sc as plsc

L = 16
NC = 2
NS = 16
NW = NC * NS
SG = 4
QW = 4
NT = 1 << QW


def _make_sc_call(B, CTX, M, LOCAL):
    BW = B // NW
    NG = BW // SG
    NQ = CTX // QW
    MH = M // 2
    MBH = MH // L
    TSTRIDE = NQ * MH
    SPG = 2 * SG

    def body(ctx_hbm, ppa_hbm, ppb_hbm, ip_hbm, out_hbm,
             ctx_v, tq_v, pp_v, ip_v, out_v):
        wid = lax.axis_index("c") * NS + lax.axis_index("s")
        base = wid * BW
        pltpu.sync_copy(ctx_hbm.at[pl.ds(base, BW)], ctx_v)
        pltpu.sync_copy(ip_hbm, ip_v)
        lane_iota = lax.iota(jnp.int32, L)
        iota4 = lane_iota * QW

        def pack_body(s, carry):
            rowvec = jnp.broadcast_to(s, (L,))
            for qc in range(NQ // L):
                g = []
                for i in range(QW):
                    cvec = iota4 + (qc * L * QW + i)
                    g.append(plsc.load_gather(ctx_v, [rowvec, cvec]))
                t = ((g[0] * 2 + g[1]) * 2 + g[2]) * 2 + g[3]
                tq_v[s, pl.ds(qc * L, L)] = t
            return carry

        lax.fori_loop(0, BW, pack_body, 0)

        for half in range(2):
            pltpu.sync_copy(ppa_hbm if half == 0 else ppb_hbm, pp_v)
            ip_rows = [[ip_v[l, pl.ds(half * MH + j * L, L)]
                        for j in range(MBH)] for l in range(LOCAL)]

            def group_body(g, carry):
                r0 = g * SG

                def chunk_body(qc, acc):
                    q0 = qc * L
                    tqrows = [tq_v[r0 + s, pl.ds(q0, L)] for s in range(SG)]
                    acc = list(acc)
                    for k in range(L):
                        qvec = lane_iota + (q0 + k) * MH
                        for s in range(SG):
                            tb = jnp.broadcast_to(tqrows[s][k], (L,))
                            ib = tb * TSTRIDE + qvec
                            for j in range(MBH):
                                row = plsc.load_gather(pp_v, [ib + j * L])
                                acc[s * MBH + j] = acc[s * MBH + j] * row
                    return tuple(acc)

                init = tuple(jnp.full((L,), 1.0, jnp.float32)
                             for _ in range(SG * MBH))
                acc = lax.fori_loop(0, NQ // L, chunk_body, init)

                carry = list(carry)
                for s in range(SG):
                    for l in range(LOCAL):
                        lane = ((g % (SPG // SG)) * SG + s) * LOCAL + l
                        v = acc[s * MBH] * ip_rows[l][0]
                        for j in range(1, MBH):
                            v = v + acc[s * MBH + j] * ip_rows[l][j]
                        for sh in (8, 4, 2, 1):
                            perm = lane_iota ^ sh
                            v = v + v.at[perm].get(mode="promise_in_bounds")
                        carry[l] = jnp.where(lane_iota == lane, v, carry[l])

                @pl.when(g % (SPG // SG) == (SPG // SG) - 1)
                def _():
                    col = (g // (SPG // SG)) * L
                    for l in range(LOCAL):
                        if half == 0:
                            out_v[pl.ds(col, L)] = carry[l] if l == 0 else (
                                out_v[pl.ds(col, L)] + carry[l])
                        else:
                            out_v[pl.ds(col, L)] = (
                                out_v[pl.ds(col, L)] + carry[l])

                return tuple(carry)

            zero = jnp.zeros((L,), jnp.float32)
            lax.fori_loop(0, NG, group_body, (zero,) * LOCAL)

        pltpu.sync_copy(out_v, out_hbm.at[pl.ds(base * LOCAL, BW * LOCAL)])

    mesh = plsc.VectorSubcoreMesh(core_axis_name="c", subcore_axis_name="s")
    return pl.kernel(
        body,
        mesh=mesh,
        compiler_params=pltpu.CompilerParams(needs_layout_passes=False),
        out_type=jax.ShapeDtypeStruct((B * LOCAL,), jnp.float32),
        scratch_types=[
            pltpu.VMEM((BW, CTX), jnp.int32),
            pltpu.VMEM((BW, NQ), jnp.int32),
            pltpu.VMEM((NT * NQ * MH,), jnp.float32),
            pltpu.VMEM((LOCAL, M), jnp.float32),
            pltpu.VMEM((BW * LOCAL,), jnp.float32),
        ],
    )


def kernel(context, context_param, inputs_param):
    LOCAL_N, M, CTX = context_param.shape
    B = context.shape[0]
    NQ = CTX // QW

    cpT = jnp.transpose(context_param, (0, 2, 1))
    gq = cpT.reshape(LOCAL_N, NQ, QW, M)
    t_idx = jnp.arange(NT)
    pp = (gq[(t_idx >> 3) & 1, :, 0, :]
          * gq[(t_idx >> 2) & 1, :, 1, :]
          * gq[(t_idx >> 1) & 1, :, 2, :]
          * gq[t_idx & 1, :, 3, :])
    ppa = pp[:, :, : M // 2].reshape(-1)
    ppb = pp[:, :, M // 2:].reshape(-1)

    call = _make_sc_call(B, CTX, M, LOCAL_N)
    out = call(context.astype(jnp.int32), ppa, ppb,
               inputs_param.astype(jnp.float32))
    return out.reshape(B, LOCAL_N)


# speedup vs baseline: 1.1140x; 1.0294x over previous
import functools

import jax
import jax.numpy as jnp
from jax import lax
from jax.experimental import pallas as pl
from jax.experimental.pallas import tpu as pltpu
from jax.experimental.pallas import tpu_sc as plsc

L = 16
NC = 2
NS = 16
NW = NC * NS
SG = 4
QW = 4
NT = 1 << QW


def _make_sc_call(B, CTX, M, LOCAL):
    BW = B // NW
    NG = BW // SG
    NQ = CTX // QW
    MH = M // 2
    MBH = MH // L
    TSTRIDE = NQ * MH
    SPG = 2 * SG

    def body(ctx_hbm, ppa_hbm, ppb_hbm, ip_hbm, out_hbm,
             ctx_v, tq_v, pp_v, ip_v, out_v):
        wid = lax.axis_index("c") * NS + lax.axis_index("s")
        base = wid * BW
        pltpu.sync_copy(ctx_hbm.at[pl.ds(base, BW)], ctx_v)
        pltpu.sync_copy(ip_hbm, ip_v)
        lane_iota = lax.iota(jnp.int32, L)
        iota4 = lane_iota * QW

        cvecs = [[iota4 + (qc * L * QW + i) for i in range(QW)]
                 for qc in range(NQ // L)]

        def pack_body(s2, carry):
            for u in range(2):
                s = s2 * 2 + u
                rowvec = jnp.broadcast_to(s, (L,))
                for qc in range(NQ // L):
                    g = [plsc.load_gather(ctx_v, [rowvec, cvecs[qc][i]])
                         for i in range(QW)]
                    t = ((g[0] * 2 + g[1]) * 2 + g[2]) * 2 + g[3]
                    tq_v[s, pl.ds(qc * L, L)] = t
            return carry

        lax.fori_loop(0, BW // 2, pack_body, 0)

        for half in range(2):
            pltpu.sync_copy(ppa_hbm if half == 0 else ppb_hbm, pp_v)
            ip_rows = [[ip_v[l, pl.ds(half * MH + j * L, L)]
                        for j in range(MBH)] for l in range(LOCAL)]

            def group_body(g, carry):
                r0 = g * SG

                def chunk_body(qc, acc):
                    q0 = qc * L
                    tqrows = [tq_v[r0 + s, pl.ds(q0, L)] for s in range(SG)]
                    acc = list(acc)
                    for k in range(L):
                        qvec = lane_iota + (q0 + k) * MH
                        for s in range(SG):
                            tb = jnp.broadcast_to(tqrows[s][k], (L,))
                            ib = tb * TSTRIDE + qvec
                            for j in range(MBH):
                                row = plsc.load_gather(pp_v, [ib + j * L])
                                acc[s * MBH + j] = acc[s * MBH + j] * row
                    return tuple(acc)

                init = tuple(jnp.full((L,), 1.0, jnp.float32)
                             for _ in range(SG * MBH))
                acc = lax.fori_loop(0, NQ // L, chunk_body, init)

                carry = list(carry)
                for s in range(SG):
                    for l in range(LOCAL):
                        lane = ((g % (SPG // SG)) * SG + s) * LOCAL + l
                        v = acc[s * MBH] * ip_rows[l][0]
                        for j in range(1, MBH):
                            v = v + acc[s * MBH + j] * ip_rows[l][j]
                        for sh in (8, 4, 2, 1):
                            perm = lane_iota ^ sh
                            v = v + v.at[perm].get(mode="promise_in_bounds")
                        carry[l] = jnp.where(lane_iota == lane, v, carry[l])

                @pl.when(g % (SPG // SG) == (SPG // SG) - 1)
                def _():
                    col = (g // (SPG // SG)) * L
                    for l in range(LOCAL):
                        if half == 0:
                            out_v[pl.ds(col, L)] = carry[l] if l == 0 else (
                                out_v[pl.ds(col, L)] + carry[l])
                        else:
                            out_v[pl.ds(col, L)] = (
                                out_v[pl.ds(col, L)] + carry[l])

                return tuple(carry)

            zero = jnp.zeros((L,), jnp.float32)
            lax.fori_loop(0, NG, group_body, (zero,) * LOCAL)

        pltpu.sync_copy(out_v, out_hbm.at[pl.ds(base * LOCAL, BW * LOCAL)])

    mesh = plsc.VectorSubcoreMesh(core_axis_name="c", subcore_axis_name="s")
    return pl.kernel(
        body,
        mesh=mesh,
        compiler_params=pltpu.CompilerParams(needs_layout_passes=False),
        out_type=jax.ShapeDtypeStruct((B * LOCAL,), jnp.float32),
        scratch_types=[
            pltpu.VMEM((BW, CTX), jnp.int32),
            pltpu.VMEM((BW, NQ), jnp.int32),
            pltpu.VMEM((NT * NQ * MH,), jnp.float32),
            pltpu.VMEM((LOCAL, M), jnp.float32),
            pltpu.VMEM((BW * LOCAL,), jnp.float32),
        ],
    )


def kernel(context, context_param, inputs_param):
    LOCAL_N, M, CTX = context_param.shape
    B = context.shape[0]
    NQ = CTX // QW

    cpT = jnp.transpose(context_param, (0, 2, 1))
    gq = cpT.reshape(LOCAL_N, NQ, QW, M)
    t_idx = jnp.arange(NT)
    pp = (gq[(t_idx >> 3) & 1, :, 0, :]
          * gq[(t_idx >> 2) & 1, :, 1, :]
          * gq[(t_idx >> 1) & 1, :, 2, :]
          * gq[t_idx & 1, :, 3, :])
    ppa = pp[:, :, : M // 2].reshape(-1)
    ppb = pp[:, :, M // 2:].reshape(-1)

    call = _make_sc_call(B, CTX, M, LOCAL_N)
    out = call(context.astype(jnp.int32), ppa, ppb,
               inputs_param.astype(jnp.float32))
    return out.reshape(B, LOCAL_N)
